# Initial kernel scaffold; baseline (speedup 1.0000x reference)
import functools

import jax
import jax.numpy as jnp
from jax import lax
from jax.experimental import pallas as pl
from jax.experimental.pallas import tpu as pltpu
from jax.experimental.pallas import tpu_sc as plsc

R = 128
C = 32768
K = 64
L = 16
NC, NS = 2, 16
NW = NC * NS
RPW = R // NW
NV = C // L

_SIGN = jnp.uint32(0x80000000)


def _keys(v):
    u = plsc.bitcast(v, jnp.uint32)
    return jnp.where(u >= _SIGN, ~u, u | _SIGN)


def _sc_body(scores_hbm, out_hbm, row_v, cand_v, tm_v):
    wid = lax.axis_index("s") * NC + lax.axis_index("c")
    lanes = lax.iota(jnp.int32, 16)
    ninf = jnp.full((L,), -jnp.inf, jnp.float32)
    res = jnp.zeros((L,), jnp.float32)

    for j in range(RPW):
        r = wid * RPW + j
        pltpu.sync_copy(scores_hbm.at[r], row_v)

        def p1(i, carry):
            pools = list(carry)
            for h in range(2):
                v = row_v[pl.ds((2 * i + h) * L, L)]
                m1, m2, m3, m4 = pools[4 * h:4 * h + 4]
                a = jnp.minimum(m1, v); m1 = jnp.maximum(m1, v)
                b = jnp.minimum(m2, a); m2 = jnp.maximum(m2, a)
                c2 = jnp.minimum(m3, b); m3 = jnp.maximum(m3, b)
                m4 = jnp.maximum(m4, c2)
                pools[4 * h:4 * h + 4] = [m1, m2, m3, m4]
            return tuple(pools)

        pools = lax.fori_loop(0, NV // 2, p1, (ninf,) * 8)
        c_lo = jnp.maximum(jnp.min(pools[3]), jnp.min(pools[7]))
        cth = jnp.full((L,), c_lo, jnp.float32)

        def p2(i, o):
            v = row_v[pl.ds(i * L, L)]
            m = v >= cth
            plsc.store_compressed(cand_v.at[pl.ds(o, L)], _keys(v), mask=m)
            return o + jnp.max(plsc.all_reduce_population_count(m))

        n = lax.fori_loop(0, NV, p2, jnp.int32(0))
        cand_v[pl.ds(n, L)] = jnp.zeros((L,), jnp.uint32)
        nv = (n + L - 1) // L

        def bitstep(b, t):
            tp = t | (jnp.uint32(1) << (jnp.uint32(31) - b.astype(jnp.uint32)))
            tpv = jnp.full((L,), tp, jnp.uint32)

            def cstep(jv, acc):
                kv = cand_v[pl.ds(jv * L, L)]
                return acc + jnp.where(kv >= tpv, 1, 0).astype(jnp.int32)

            acc = lax.fori_loop(0, nv, cstep, jnp.zeros((L,), jnp.int32))
            return jnp.where(jnp.sum(acc) >= K, tp, t)

        tkey = lax.fori_loop(0, 32, bitstep, jnp.uint32(0))
        u = jnp.where(tkey >= _SIGN, tkey ^ _SIGN, ~tkey)
        thr = lax.bitcast_convert_type(u, jnp.float32)
        res = jnp.where(lanes == j, jnp.full((L,), thr, jnp.float32), res)

    tm_v[...] = res
    pltpu.sync_copy(tm_v, out_hbm.at[wid])


@jax.jit
def _sc_thresholds(scores):
    mesh = plsc.VectorSubcoreMesh(
        core_axis_name="c", subcore_axis_name="s", num_cores=NC, num_subcores=NS)
    f = pl.kernel(
        _sc_body,
        out_type=jax.ShapeDtypeStruct((NW, L), jnp.float32),
        mesh=mesh,
        scratch_types=[
            pltpu.VMEM((C,), jnp.float32),
            pltpu.VMEM((C + L,), jnp.uint32),
            pltpu.VMEM((L,), jnp.float32),
        ],
    )
    return f(scores)


def _tc_body(s_ref, t_ref, o_ref):
    s = s_ref[...]
    t = t_ref[...]
    masked = jnp.where(s >= t, s, jnp.float32(-1e9))
    m = jnp.max(masked, axis=-1, keepdims=True)
    e = jnp.exp(masked - m)
    z = jnp.sum(e, axis=-1, keepdims=True)
    o_ref[...] = e / z


@functools.partial(jax.jit, static_argnames=("block_r",))
def _tc_softmax(scores, thresh, block_r=8):
    return pl.pallas_call(
        _tc_body,
        grid=(R // block_r,),
        in_specs=[
            pl.BlockSpec((block_r, C), lambda i: (i, 0)),
            pl.BlockSpec((block_r, 1), lambda i: (i, 0)),
        ],
        out_specs=pl.BlockSpec((block_r, C), lambda i: (i, 0)),
        out_shape=jax.ShapeDtypeStruct((R, C), jnp.float32),
    )(scores, thresh)


def kernel(scores, k):
    del k
    tm = _sc_thresholds(scores)
    thresh = tm[:, :RPW].reshape(R, 1)
    return _tc_softmax(scores, thresh)


# speedup vs baseline: 5.9222x; 5.9222x over previous
import functools

import jax
import jax.numpy as jnp
import numpy as np
from jax import lax
from jax.experimental import pallas as pl
from jax.experimental.pallas import tpu as pltpu
from jax.experimental.pallas import tpu_sc as plsc

R = 128
C = 32768
K = 64
L = 16
NC, NS = 2, 16
NW = NC * NS
RPW = R // NW
NV = C // L

_SIGN = np.uint32(0x80000000)


def _keys(v):
    u = plsc.bitcast(v, jnp.uint32)
    return jnp.where(u >= _SIGN, ~u, u | _SIGN)


_GDN = lax.GatherDimensionNumbers(
    offset_dims=(), collapsed_slice_dims=(0,), start_index_map=(0,))


def _shuf(x, idx):
    return lax.gather(x, idx[:, None], _GDN, slice_sizes=(1,),
                      mode=lax.GatherScatterMode.PROMISE_IN_BOUNDS)


def _lane_reduce(x, op, lanes):
    for s in (8, 4, 2, 1):
        x = op(x, _shuf(x, lanes ^ s))
    return x


def _sc_body(scores_hbm, out_hbm, row_v, cand_v, tm_v):
    wid = lax.axis_index("s") * NC + lax.axis_index("c")
    lanes = lax.iota(jnp.int32, 16)
    ninf = jnp.full((L,), -jnp.inf, jnp.float32)
    res = jnp.zeros((L,), jnp.float32)

    for j in range(RPW):
        r = wid * RPW + j
        pltpu.sync_copy(scores_hbm.at[r], row_v)

        def p1(i, carry):
            pools = list(carry)
            for h in range(2):
                v = row_v[pl.ds((2 * i + h) * L, L)]
                m1, m2, m3, m4 = pools[4 * h:4 * h + 4]
                a = jnp.minimum(m1, v); m1 = jnp.maximum(m1, v)
                b = jnp.minimum(m2, a); m2 = jnp.maximum(m2, a)
                c2 = jnp.minimum(m3, b); m3 = jnp.maximum(m3, b)
                m4 = jnp.maximum(m4, c2)
                pools[4 * h:4 * h + 4] = [m1, m2, m3, m4]
            return tuple(pools)

        pools = lax.fori_loop(0, NV // 2, p1, (ninf,) * 8)
        cth = jnp.maximum(_lane_reduce(pools[3], jnp.minimum, lanes),
                          _lane_reduce(pools[7], jnp.minimum, lanes))

        def p2(i, o):
            v = row_v[pl.ds(i * L, L)]
            m = v >= cth
            plsc.store_compressed(cand_v.at[pl.ds(o, L)], _keys(v), mask=m)
            return o + plsc.all_reduce_population_count(m)[0]

        n = lax.fori_loop(0, NV, p2, jnp.int32(0))
        cand_v[pl.ds(n, L)] = jnp.zeros((L,), jnp.uint32)
        nv = (n + L - 1) // L

        def bitstep(b, t):
            bit = jnp.full((L,), 1, jnp.uint32) << (31 - b).astype(jnp.uint32)
            tp = t | bit

            def cstep(jv, acc):
                kv = cand_v[pl.ds(jv * L, L)]
                return acc + jnp.where(kv >= tp, 1, 0).astype(jnp.int32)

            acc = lax.fori_loop(0, nv, cstep, jnp.zeros((L,), jnp.int32))
            total = _lane_reduce(acc, jnp.add, lanes)
            return jnp.where(total >= K, tp, t)

        tkey = lax.fori_loop(0, 32, bitstep, jnp.zeros((L,), jnp.uint32))
        u = jnp.where(tkey >= _SIGN, tkey ^ _SIGN, ~tkey)
        thr = plsc.bitcast(u, jnp.float32)
        res = jnp.where(lanes == j, thr, res)

    tm_v[...] = res
    pltpu.sync_copy(tm_v, out_hbm.at[wid])


@jax.jit
def _sc_thresholds(scores):
    mesh = plsc.VectorSubcoreMesh(
        core_axis_name="c", subcore_axis_name="s", num_cores=NC, num_subcores=NS)
    f = pl.kernel(
        _sc_body,
        out_type=jax.ShapeDtypeStruct((NW, L), jnp.float32),
        mesh=mesh,
        compiler_params=pltpu.CompilerParams(needs_layout_passes=False),
        scratch_types=[
            pltpu.VMEM((C,), jnp.float32),
            pltpu.VMEM((C + L,), jnp.uint32),
            pltpu.VMEM((L,), jnp.float32),
        ],
    )
    return f(scores)


def _tc_body(s_ref, t_ref, o_ref):
    s = s_ref[...]
    t = t_ref[...]
    masked = jnp.where(s >= t, s, jnp.float32(-1e9))
    m = jnp.max(masked, axis=-1, keepdims=True)
    e = jnp.exp(masked - m)
    z = jnp.sum(e, axis=-1, keepdims=True)
    o_ref[...] = e / z


@functools.partial(jax.jit, static_argnames=("block_r",))
def _tc_softmax(scores, thresh, block_r=8):
    return pl.pallas_call(
        _tc_body,
        grid=(R // block_r,),
        in_specs=[
            pl.BlockSpec((block_r, C), lambda i: (i, 0)),
            pl.BlockSpec((block_r, 1), lambda i: (i, 0)),
        ],
        out_specs=pl.BlockSpec((block_r, C), lambda i: (i, 0)),
        out_shape=jax.ShapeDtypeStruct((R, C), jnp.float32),
    )(scores, thresh)


def kernel(scores, k):
    del k
    tm = _sc_thresholds(scores)
    thresh = tm[:, :RPW].reshape(R, 1)
    return _tc_softmax(scores, thresh)
